# baseline (speedup 1.0000x reference)
import functools
import math

import jax
import jax.numpy as jnp
import numpy as np
from jax import lax
from jax.experimental import pallas as pl
from jax.experimental.pallas import tpu as pltpu
from jax.experimental.pallas import tpu_sc as plsc

VOCAB = 1000000
EMB = 64
B = 4096
S = 200
SCALE = math.sqrt(EMB)

_info = plsc.get_sparse_core_info()
NC, NS, L = _info.num_cores, _info.num_subcores, _info.num_lanes
NW = NC * NS
BW = B // NW
KB = BW // L
_ROW_SLICES = [16 * j for j in range(S // 16)] + [S - 16]


def _pos_encoding_np(max_len, d):
    pos = np.arange(max_len)[:, None].astype(np.float32)
    div = np.exp(np.arange(0, d, 2).astype(np.float32) * (-math.log(10000.0) / d))
    pe = np.zeros((max_len, d), dtype=np.float32)
    pe[:, 0::2] = np.sin(pos * div)
    pe[:, 1::2] = np.cos(pos * div)
    return pe


_PE_NP = _pos_encoding_np(S, EMB)


def _body(tok_hbm, xf_hbm, pe_hbm, out_hbm, pe_v, idxf_v, idx_v, rows, ovt, gsems, ssems):
    wid = lax.axis_index("s") * NC + lax.axis_index("c")
    b0 = wid * BW

    pltpu.sync_copy(pe_hbm, pe_v)
    pltpu.sync_copy(xf_hbm.at[pl.ds(b0, BW)], idxf_v)

    iota16 = lax.iota(jnp.int32, L)

    def brow(s, c):
        s16 = jnp.full((L,), s, jnp.int32)
        for k in range(KB):
            v = plsc.load_gather(idxf_v, [iota16 + k * L, s16])
            idx_v[s, pl.ds(k * L, L)] = plsc.bitcast(v, jnp.int32)
        return c

    lax.fori_loop(0, S, brow, 0)

    def fire_gather(s, p):
        pltpu.async_copy(tok_hbm.at[idx_v.at[s, pl.ds(0, BW)]], rows[p], gsems[p])

    def wait_gather(s, p):
        pltpu.make_async_copy(
            tok_hbm.at[idx_v.at[s, pl.ds(0, BW)]], rows[p], gsems[p]
        ).wait()

    def _out_slice(s):
        return out_hbm.at[pl.ds(s * EMB, EMB), pl.ds(b0, BW)]

    def fire_scatter(s, p):
        pltpu.async_copy(ovt[p], _out_slice(s), ssems[p])

    def wait_scatter(s, p):
        pltpu.make_async_copy(ovt[p], _out_slice(s), ssems[p]).wait()

    def compute(s, p):
        rv = rows[p]
        ov = ovt[p]
        s16 = jnp.full((L,), s, jnp.int32)

        def crow(e, c):
            e16 = jnp.full((L,), e, jnp.int32)
            peb = plsc.load_gather(pe_v, [s16, e16])
            for k in range(KB):
                g = plsc.load_gather(rv, [iota16 + k * L, e16])
                ov[e, pl.ds(k * L, L)] = g * SCALE + peb
            return c

        lax.fori_loop(0, EMB, crow, 0)

    def step(s, p, first=False, last=False):
        if not first:
            wait_scatter(s - 1, 1 - p)
        if not last:
            fire_gather(s + 1, 1 - p)
        wait_gather(s, p)
        compute(s, p)
        fire_scatter(s, p)

    fire_gather(0, 0)
    step(0, 0, first=True)

    def pair(k, c):
        step(2 * k + 1, 1)
        step(2 * k + 2, 0)
        return c

    lax.fori_loop(0, (S - 2) // 2, pair, 0)
    step(S - 1, 1, last=True)
    wait_scatter(S - 1, 1)


@jax.jit
def _emb_lookup(tok_emb, xf32, pe):
    mesh = plsc.VectorSubcoreMesh(core_axis_name="c", subcore_axis_name="s")
    f = pl.kernel(
        _body,
        mesh=mesh,
        out_type=jax.ShapeDtypeStruct((S * EMB, B), jnp.float32),
        scratch_types=[
            pltpu.VMEM((S, EMB), jnp.float32),
            pltpu.VMEM((BW, S), jnp.float32),
            pltpu.VMEM((S, BW), jnp.int32),
            [pltpu.VMEM((BW, EMB), jnp.float32) for _ in range(2)],
            [pltpu.VMEM((EMB, BW), jnp.float32) for _ in range(2)],
            [pltpu.SemaphoreType.DMA for _ in range(2)],
            [pltpu.SemaphoreType.DMA for _ in range(2)],
        ],
        compiler_params=pltpu.CompilerParams(
            use_tc_tiling_on_sc=False, needs_layout_passes=False
        ),
    )
    return f(tok_emb, xf32, pe)


def kernel(x, tok_emb):
    pe = jnp.asarray(_PE_NP)
    xf32 = jax.lax.bitcast_convert_type(x.astype(jnp.int32), jnp.float32)
    flat = _emb_lookup(tok_emb, xf32, pe)
    return flat.reshape(S, EMB, B).transpose(2, 0, 1)


# speedup vs baseline: 1.5616x; 1.5616x over previous
import functools
import math

import jax
import jax.numpy as jnp
import numpy as np
from jax import lax
from jax.experimental import pallas as pl
from jax.experimental.pallas import tpu as pltpu
from jax.experimental.pallas import tpu_sc as plsc

VOCAB = 1000000
EMB = 64
B = 4096
S = 200
SCALE = math.sqrt(EMB)

_info = plsc.get_sparse_core_info()
NC, NS, L = _info.num_cores, _info.num_subcores, _info.num_lanes
NW = NC * NS
BW = B // NW
KB = BW // L
N_VREG = EMB // L
_ROW_SLICES = [16 * j for j in range(S // 16)] + [S - 16]


def _pos_encoding_np(max_len, d):
    pos = np.arange(max_len)[:, None].astype(np.float32)
    div = np.exp(np.arange(0, d, 2).astype(np.float32) * (-math.log(10000.0) / d))
    pe = np.zeros((max_len, d), dtype=np.float32)
    pe[:, 0::2] = np.sin(pos * div)
    pe[:, 1::2] = np.cos(pos * div)
    return pe


_PE_NP = _pos_encoding_np(S, EMB)


def _body(tok_hbm, xf_hbm, pe_hbm, out_hbm, pe_v, idxf_v, idx_v, rows, ovt, gsems, ssems):
    wid = lax.axis_index("s") * NC + lax.axis_index("c")
    b0 = wid * BW

    pltpu.sync_copy(pe_hbm, pe_v)
    pltpu.sync_copy(xf_hbm.at[pl.ds(b0, BW)], idxf_v)

    iota16 = lax.iota(jnp.int32, L)

    def brow(s, c):
        s16 = jnp.full((L,), s, jnp.int32)
        for k in range(KB):
            v = plsc.load_gather(idxf_v, [iota16 + k * L, s16])
            idx_v[s, pl.ds(k * L, L)] = plsc.bitcast(v, jnp.int32)
        return c

    lax.fori_loop(0, S, brow, 0)

    def fire_gather(s, p):
        pltpu.async_copy(tok_hbm.at[idx_v.at[s, pl.ds(0, BW)]], rows[p], gsems[p])

    def wait_gather(s, p):
        pltpu.make_async_copy(
            tok_hbm.at[idx_v.at[s, pl.ds(0, BW)]], rows[p], gsems[p]
        ).wait()

    def _out_slice(s):
        return out_hbm.at[pl.ds(s * EMB, EMB), pl.ds(b0, BW)]

    def fire_scatter(s, p):
        pltpu.async_copy(ovt[p].at[:, pl.ds(0, BW)], _out_slice(s), ssems[p])

    def wait_scatter(s, p):
        pltpu.make_async_copy(
            ovt[p].at[:, pl.ds(0, BW)], _out_slice(s), ssems[p]
        ).wait()

    def compute(s, p):
        rv = rows[p]
        ov = ovt[p]
        pes = [pe_v[s, pl.ds(j * L, L)] for j in range(N_VREG)]

        def crow(b, c):
            b16 = jnp.full((L,), b, jnp.int32)
            for j in range(N_VREG):
                g = rv[b, pl.ds(j * L, L)]
                plsc.store_scatter(ov, [iota16 + j * L, b16], g * SCALE + pes[j])
            return c

        lax.fori_loop(0, BW, crow, 0)

    def step(s, p, first=False, last=False):
        if not first:
            wait_scatter(s - 1, 1 - p)
        if not last:
            fire_gather(s + 1, 1 - p)
        wait_gather(s, p)
        compute(s, p)
        fire_scatter(s, p)

    fire_gather(0, 0)
    step(0, 0, first=True)

    def pair(k, c):
        step(2 * k + 1, 1)
        step(2 * k + 2, 0)
        return c

    lax.fori_loop(0, (S - 2) // 2, pair, 0)
    step(S - 1, 1, last=True)
    wait_scatter(S - 1, 1)


@jax.jit
def _emb_lookup(tok_emb, xf32, pe):
    mesh = plsc.VectorSubcoreMesh(core_axis_name="c", subcore_axis_name="s")
    f = pl.kernel(
        _body,
        mesh=mesh,
        out_type=jax.ShapeDtypeStruct((S * EMB, B), jnp.float32),
        scratch_types=[
            pltpu.VMEM((S, EMB), jnp.float32),
            pltpu.VMEM((BW, S), jnp.float32),
            pltpu.VMEM((S, BW), jnp.int32),
            [pltpu.VMEM((BW, EMB), jnp.float32) for _ in range(2)],
            [pltpu.VMEM((EMB, BW + 1), jnp.float32) for _ in range(2)],
            [pltpu.SemaphoreType.DMA for _ in range(2)],
            [pltpu.SemaphoreType.DMA for _ in range(2)],
        ],
        compiler_params=pltpu.CompilerParams(
            use_tc_tiling_on_sc=False, needs_layout_passes=False
        ),
    )
    return f(tok_emb, xf32, pe)


def kernel(x, tok_emb):
    pe = jnp.asarray(_PE_NP)
    xf32 = jax.lax.bitcast_convert_type(x.astype(jnp.int32), jnp.float32)
    flat = _emb_lookup(tok_emb, xf32, pe)
    return flat.reshape(S, EMB, B).transpose(2, 0, 1)


# speedup vs baseline: 1.8251x; 1.1687x over previous
import functools
import math

import jax
import jax.numpy as jnp
import numpy as np
from jax import lax
from jax.experimental import pallas as pl
from jax.experimental.pallas import tpu as pltpu
from jax.experimental.pallas import tpu_sc as plsc

VOCAB = 1000000
EMB = 64
B = 4096
S = 200
SCALE = math.sqrt(EMB)

_info = plsc.get_sparse_core_info()
NC, NS, L = _info.num_cores, _info.num_subcores, _info.num_lanes
NW = NC * NS
SEQ_PER_W = B // NW
N_VREG = EMB // L
G1 = 128
G2 = S - G1
ROWS_W = SEQ_PER_W * S
OC = 128
OROW_SEQ = S * EMB // OC
_ROW_SLICES = [16 * j for j in range(S // 16)] + [S - 16]


def _pos_encoding_np(max_len, d):
    pos = np.arange(max_len)[:, None].astype(np.float32)
    div = np.exp(np.arange(0, d, 2).astype(np.float32) * (-math.log(10000.0) / d))
    pe = np.zeros((max_len, d), dtype=np.float32)
    pe[:, 0::2] = np.sin(pos * div)
    pe[:, 1::2] = np.cos(pos * div)
    return pe


_PE_NP = _pos_encoding_np(S, EMB)


def _body(tok_hbm, xf_hbm, pe_hbm, out_hbm, pe_v, idxf_v, idx_v, rows, oflat, gsems, ssems):
    wid = lax.axis_index("s") * NC + lax.axis_index("c")
    seq0 = wid * SEQ_PER_W
    base = wid * SEQ_PER_W * OROW_SEQ

    pltpu.sync_copy(pe_hbm, pe_v)
    pltpu.sync_copy(xf_hbm.at[pl.ds(seq0, SEQ_PER_W)], idxf_v)

    def brow(r, c):
        for off in _ROW_SLICES:
            sl = pl.ds(off, L)
            idx_v[r, sl] = plsc.bitcast(idxf_v[r, sl], jnp.int32)
        return c

    lax.fori_loop(0, SEQ_PER_W, brow, 0)

    def fire_gather(i, p):
        pltpu.async_copy(
            tok_hbm.at[idx_v.at[i, pl.ds(0, G1)]], rows[p].at[pl.ds(0, G1)], gsems[p]
        )
        pltpu.async_copy(
            tok_hbm.at[idx_v.at[i, pl.ds(G1, G2)]], rows[p].at[pl.ds(G1, G2)], gsems[p]
        )

    def wait_gather(i, p):
        pltpu.make_async_copy(
            tok_hbm.at[idx_v.at[i, pl.ds(0, G1)]], rows[p].at[pl.ds(0, G1)], gsems[p]
        ).wait()
        pltpu.make_async_copy(
            tok_hbm.at[idx_v.at[i, pl.ds(G1, G2)]], rows[p].at[pl.ds(G1, G2)], gsems[p]
        ).wait()

    def _out_slice(i):
        return out_hbm.at[pl.ds(base + i * OROW_SEQ, OROW_SEQ)]

    def fire_scatter(i, p):
        pltpu.async_copy(oflat[p], _out_slice(i), ssems[p])

    def wait_scatter(i, p):
        pltpu.make_async_copy(oflat[p], _out_slice(i), ssems[p]).wait()

    def compute(p):
        rv = rows[p]
        ov = oflat[p]

        def crow(r, c):
            for u in range(2):
                rr = 2 * r + u
                for j in range(N_VREG):
                    sl = pl.ds(j * L, L)
                    ov[r, pl.ds((4 * u + j) * L, L)] = rv[rr, sl] * SCALE + pe_v[rr, sl]
            return c

        lax.fori_loop(0, S // 2, crow, 0)

    def step(i, p, first=False, last=False):
        if not first:
            wait_scatter(i - 1, 1 - p)
        if not last:
            fire_gather(i + 1, 1 - p)
        wait_gather(i, p)
        compute(p)
        fire_scatter(i, p)

    fire_gather(0, 0)
    step(0, 0, first=True)

    def pair(k, c):
        step(2 * k + 1, 1)
        step(2 * k + 2, 0)
        return c

    lax.fori_loop(0, (SEQ_PER_W - 2) // 2, pair, 0)
    step(SEQ_PER_W - 1, 1, last=True)
    wait_scatter(SEQ_PER_W - 1, 1)


@jax.jit
def _emb_lookup(tok_emb, xf32, pe):
    mesh = plsc.VectorSubcoreMesh(core_axis_name="c", subcore_axis_name="s")
    f = pl.kernel(
        _body,
        mesh=mesh,
        out_type=jax.ShapeDtypeStruct((B * S * EMB // OC, OC), jnp.float32),
        scratch_types=[
            pltpu.VMEM((S, EMB), jnp.float32),
            pltpu.VMEM((SEQ_PER_W, S), jnp.float32),
            pltpu.VMEM((SEQ_PER_W, S), jnp.int32),
            [pltpu.VMEM((S, EMB), jnp.float32) for _ in range(2)],
            [pltpu.VMEM((OROW_SEQ, OC), jnp.float32) for _ in range(2)],
            [pltpu.SemaphoreType.DMA for _ in range(2)],
            [pltpu.SemaphoreType.DMA for _ in range(2)],
        ],
        compiler_params=pltpu.CompilerParams(
            use_tc_tiling_on_sc=False, needs_layout_passes=False
        ),
    )
    return f(tok_emb, xf32, pe)


def kernel(x, tok_emb):
    pe = jnp.asarray(_PE_NP)
    xf32 = jax.lax.bitcast_convert_type(x.astype(jnp.int32), jnp.float32)
    flat = _emb_lookup(tok_emb, xf32, pe)
    return flat.reshape(B, S, EMB)
